# Initial kernel scaffold; baseline (speedup 1.0000x reference)
import jax
import jax.numpy as jnp
from jax.experimental import pallas as pl


def kernel(x, W, b):
    raise NotImplementedError("write your pallas kernel here")


# speedup vs baseline: 1.6655x; 1.6655x over previous
import functools

import jax
import jax.numpy as jnp
from jax.experimental import pallas as pl

N_NEI = 4
D = 2048
NV = 20
B = 32
M = B * NV


def _matmul_body(x_ref, w_ref, c1_ref, c2_ref):
    xb = x_ref[...]
    wb = w_ref[...]
    w1 = wb[:, :D]
    w2 = wb[:, D:]
    dn = (((1,), (1,)), ((), ()))
    c1_ref[...] = jax.lax.dot_general(xb, w1, dn,
                                      preferred_element_type=jnp.float32)
    c2_ref[...] = jax.lax.dot_general(xb, w2, dn,
                                      preferred_element_type=jnp.float32)


def _combine_body(x_ref, c1_ref, c2_ref, b_ref, out_ref):
    xb = x_ref[0]
    c1 = c1_ref[0]
    c2 = c2_ref[0]
    bias = b_ref[...]

    gram = jax.lax.dot_general(xb, xb, (((1,), (1,)), ((), ())),
                               preferred_element_type=jnp.float32)
    sq = jnp.sum(xb * xb, axis=1)
    inner = -2.0 * gram
    adj = (sq[None, :] + inner) + sq[:, None]

    lane = jax.lax.broadcasted_iota(jnp.int32, (NV, NV), 1)
    cur = adj
    p = None
    for k in range(N_NEI):
        mn = jnp.min(cur, axis=1, keepdims=True)
        eq = cur == mn
        idx = jnp.min(jnp.where(eq, lane, NV), axis=1, keepdims=True)
        oh = lane == idx
        pk = jax.lax.dot_general(oh.astype(jnp.float32), c2,
                                 (((1,), (0,)), ((), ())),
                                 preferred_element_type=jnp.float32)
        p = pk if p is None else jnp.maximum(p, pk)
        if k + 1 < N_NEI:
            cur = jnp.where(oh, jnp.float32(jnp.inf), cur)

    h = jnp.maximum(c1 - c2 + p + bias, 0.0)
    out_ref[0, 0, :] = jnp.max(h, axis=0)


@functools.partial(jax.jit, static_argnames=())
def kernel(x, W, b):
    x2d = x.reshape(M, D)
    x_bf = x2d.astype(jnp.bfloat16)
    w_bf = W.astype(jnp.bfloat16)

    bn = 512
    c1, c2 = pl.pallas_call(
        _matmul_body,
        grid=(D // bn,),
        in_specs=[
            pl.BlockSpec((M, D), lambda j: (0, 0)),
            pl.BlockSpec((bn, 2 * D), lambda j: (j, 0)),
        ],
        out_specs=[
            pl.BlockSpec((M, bn), lambda j: (0, j)),
            pl.BlockSpec((M, bn), lambda j: (0, j)),
        ],
        out_shape=[
            jax.ShapeDtypeStruct((M, D), jnp.float32),
            jax.ShapeDtypeStruct((M, D), jnp.float32),
        ],
    )(x_bf, w_bf)

    c1r = c1.reshape(B, NV, D)
    c2r = c2.reshape(B, NV, D)
    b2d = b.reshape(1, D)

    out = pl.pallas_call(
        _combine_body,
        grid=(B,),
        in_specs=[
            pl.BlockSpec((1, NV, D), lambda i: (i, 0, 0)),
            pl.BlockSpec((1, NV, D), lambda i: (i, 0, 0)),
            pl.BlockSpec((1, NV, D), lambda i: (i, 0, 0)),
            pl.BlockSpec((1, D), lambda i: (0, 0)),
        ],
        out_specs=pl.BlockSpec((1, 1, D), lambda i: (i, 0, 0)),
        out_shape=jax.ShapeDtypeStruct((B, 1, D), jnp.float32),
    )(x, c1r, c2r, b2d)

    return out.reshape(B, D, 1, 1)


# speedup vs baseline: 2.8724x; 1.7247x over previous
import functools

import jax
import jax.numpy as jnp
from jax.experimental import pallas as pl
from jax.experimental.pallas import tpu as pltpu

N_NEI = 4
D = 2048
NV = 20
B = 32
M = B * NV
BN = 512


def _select_body(x_ref, s0_ref, s1_ref, s2_ref, s3_ref, adj_ref):
    for b in range(B):
        xb = x_ref[b]
        gram = jax.lax.dot_general(xb, xb, (((1,), (1,)), ((), ())),
                                   preferred_element_type=jnp.float32)
        sq = jnp.sum(xb * xb, axis=1)
        inner = -2.0 * gram
        adj_ref[b * NV:(b + 1) * NV, :] = (sq[None, :] + inner) + sq[:, None]

    cur = adj_ref[...]
    lane = jax.lax.broadcasted_iota(jnp.int32, (M, NV), 1)
    s_refs = (s0_ref, s1_ref, s2_ref, s3_ref)
    for k in range(N_NEI):
        mn = jnp.min(cur, axis=1, keepdims=True)
        eq = cur == mn
        idx = jnp.min(jnp.where(eq, lane, NV), axis=1, keepdims=True)
        oh = lane == idx
        s_refs[k][...] = oh.astype(jnp.float32)
        if k + 1 < N_NEI:
            cur = jnp.where(oh, jnp.float32(jnp.inf), cur)


def _fused_body(x_ref, w_ref, s0_ref, s1_ref, s2_ref, s3_ref, b_ref, out_ref):
    xb = x_ref[...]
    wb = w_ref[...]
    dn = (((1,), (1,)), ((), ()))
    c1 = jax.lax.dot_general(xb, wb[:, :D], dn,
                             preferred_element_type=jnp.float32)
    c2 = jax.lax.dot_general(xb, wb[:, D:], dn,
                             preferred_element_type=jnp.float32)
    d = c1 - c2 + b_ref[...]

    s_refs = (s0_ref, s1_ref, s2_ref, s3_ref)
    for b in range(B):
        rows = slice(b * NV, (b + 1) * NV)
        c2b = c2[rows]
        p = None
        for k in range(N_NEI):
            sk = s_refs[k][rows, :]
            pk = jax.lax.dot_general(sk, c2b, (((1,), (0,)), ((), ())),
                                     preferred_element_type=jnp.float32)
            p = pk if p is None else jnp.maximum(p, pk)
        h = jnp.maximum(d[rows] + p, 0.0)
        out_ref[b, :] = jnp.max(h, axis=0)


@functools.partial(jax.jit, static_argnames=())
def kernel(x, W, b):
    x2d = x.reshape(M, D)
    x_bf = x2d.astype(jnp.bfloat16)
    w_bf = W.astype(jnp.bfloat16)
    b2d = b.reshape(1, D)

    s0, s1, s2, s3 = pl.pallas_call(
        _select_body,
        grid=(1,),
        in_specs=[pl.BlockSpec((B, NV, D), lambda i: (0, 0, 0))],
        out_specs=[pl.BlockSpec((M, NV), lambda i: (0, 0))] * N_NEI,
        out_shape=[jax.ShapeDtypeStruct((M, NV), jnp.float32)] * N_NEI,
        scratch_shapes=[pltpu.VMEM((M, NV), jnp.float32)],
    )(x)

    out = pl.pallas_call(
        _fused_body,
        grid=(D // BN,),
        in_specs=[
            pl.BlockSpec((M, D), lambda j: (0, 0)),
            pl.BlockSpec((BN, 2 * D), lambda j: (j, 0)),
            pl.BlockSpec((M, NV), lambda j: (0, 0)),
            pl.BlockSpec((M, NV), lambda j: (0, 0)),
            pl.BlockSpec((M, NV), lambda j: (0, 0)),
            pl.BlockSpec((M, NV), lambda j: (0, 0)),
            pl.BlockSpec((1, BN), lambda j: (0, j)),
        ],
        out_specs=pl.BlockSpec((B, BN), lambda j: (0, j)),
        out_shape=jax.ShapeDtypeStruct((B, D), jnp.float32),
    )(x_bf, w_bf, s0, s1, s2, s3, b2d)

    return out.reshape(B, D, 1, 1)


# speedup vs baseline: 5.3192x; 1.8518x over previous
import functools

import jax
import jax.numpy as jnp
from jax.experimental import pallas as pl
from jax.experimental.pallas import tpu as pltpu

N_NEI = 4
D = 2048
NV = 20
B = 32
M = B * NV
BN = 512


def _fused_body(x_ref, w_ref, b_ref, out_ref, xbf_ref, adj_ref,
                s0_ref, s1_ref, s2_ref, s3_ref):
    j = pl.program_id(0)
    s_refs = (s0_ref, s1_ref, s2_ref, s3_ref)

    @pl.when(j == 0)
    def _select():
        xv = x_ref[...]
        xbf_ref[...] = xv.reshape(M, D).astype(jnp.bfloat16)
        r_ix = jax.lax.broadcasted_iota(jnp.int32, (NV, NV), 0)
        c_ix = jax.lax.broadcasted_iota(jnp.int32, (NV, NV), 1)
        eye = (r_ix == c_ix).astype(jnp.float32)
        for b in range(B):
            xb = xv[b]
            g = jax.lax.dot_general(xb, xb, (((1,), (1,)), ((), ())),
                                    preferred_element_type=jnp.float32)
            sq = jnp.sum(g * eye, axis=0)
            adj_ref[b * NV:(b + 1) * NV, :] = sq[None, :] - 2.0 * g

        cur = adj_ref[...]
        lane = jax.lax.broadcasted_iota(jnp.int32, (M, NV), 1)
        for k in range(N_NEI):
            mn = jnp.min(cur, axis=1, keepdims=True)
            eq = cur == mn
            idx = jnp.min(jnp.where(eq, lane, NV), axis=1, keepdims=True)
            oh = lane == idx
            s_refs[k][...] = oh.astype(jnp.float32)
            if k + 1 < N_NEI:
                cur = jnp.where(oh, jnp.float32(jnp.inf), cur)

    xbf = xbf_ref[...]
    wbf = w_ref[...].astype(jnp.bfloat16)
    dn = (((1,), (1,)), ((), ()))
    c1 = jax.lax.dot_general(xbf, wbf[:, :D], dn,
                             preferred_element_type=jnp.float32)
    c2 = jax.lax.dot_general(xbf, wbf[:, D:], dn,
                             preferred_element_type=jnp.float32)
    d = c1 - c2 + b_ref[...]

    for b in range(B):
        rows = slice(b * NV, (b + 1) * NV)
        c2b = c2[rows]
        p = None
        for k in range(N_NEI):
            sk = s_refs[k][rows, :]
            pk = jax.lax.dot_general(sk, c2b, (((1,), (0,)), ((), ())),
                                     preferred_element_type=jnp.float32)
            p = pk if p is None else jnp.maximum(p, pk)
        h = jnp.maximum(d[rows] + p, 0.0)
        out_ref[b, :] = jnp.max(h, axis=0)


@functools.partial(jax.jit, static_argnames=())
def kernel(x, W, b):
    b2d = b.reshape(1, D)

    out = pl.pallas_call(
        _fused_body,
        grid=(D // BN,),
        in_specs=[
            pl.BlockSpec((B, NV, D), lambda j: (0, 0, 0)),
            pl.BlockSpec((BN, 2 * D), lambda j: (j, 0)),
            pl.BlockSpec((1, BN), lambda j: (0, j)),
        ],
        out_specs=pl.BlockSpec((B, BN), lambda j: (0, j)),
        out_shape=jax.ShapeDtypeStruct((B, D), jnp.float32),
        scratch_shapes=[
            pltpu.VMEM((M, D), jnp.bfloat16),
            pltpu.VMEM((M, NV), jnp.float32),
            pltpu.VMEM((M, NV), jnp.float32),
            pltpu.VMEM((M, NV), jnp.float32),
            pltpu.VMEM((M, NV), jnp.float32),
            pltpu.VMEM((M, NV), jnp.float32),
        ],
    )(x, W, b2d)

    return out.reshape(B, D, 1, 1)
